# baseline (speedup 1.0000x reference)
import jax
import jax.numpy as jnp
from jax import lax
from jax.experimental import pallas as pl
from jax.experimental.pallas import tpu as pltpu
from jax.experimental.pallas import tpu_sc as plsc

_T_TOTAL = 100000
_OBS = 512
_H1 = 256
_H2 = 128
_ACT = 32
_NJ = 32
_NB = 16
_BATCH = 4096

_NC, _NS, _L = 2, 16, 16
_NW = _NC * _NS
_BPW = _BATCH // _NW

_VA = _T_TOTAL * _NJ // 128
_VQ = _T_TOTAL * _NB * 4 // 128
_VB = _T_TOTAL * _NB * 3 // 128

_sc_mesh = plsc.VectorSubcoreMesh(
    core_axis_name="c", subcore_axis_name="s",
    num_cores=_NC, num_subcores=_NS)


def _gather_body(ts_hbm, jp, jv, bq, bp, blv, bav,
                 o_jp, o_jv, o_bq, o_bp0, o_bp1, o_blv0, o_blv1,
                 o_bav0, o_bav1,
                 t_v, ra_v, rq_v, rb0_v, rb1_v,
                 b0, b1, b2, s0, s1, s2):
    wid = lax.axis_index("s") * _NC + lax.axis_index("c")
    base = wid * _BPW
    pltpu.sync_copy(ts_hbm.at[pl.ds(base, _BPW)], t_v)
    for i in range(_BPW // _L):
        sl = pl.ds(i * _L, _L)
        t = jnp.minimum(t_v[sl], _T_TOTAL - 1)
        ra_v[sl] = t >> 2
        rq_v[sl] = t >> 1
        pb = (t * 48) >> 7
        rb0_v[sl] = pb
        rb1_v[sl] = jnp.minimum(pb + 1, _VB - 1)
    work = ((jp, ra_v, o_jp), (jv, ra_v, o_jv), (bq, rq_v, o_bq),
            (bp, rb0_v, o_bp0), (bp, rb1_v, o_bp1),
            (blv, rb0_v, o_blv0), (blv, rb1_v, o_blv1),
            (bav, rb0_v, o_bav0), (bav, rb1_v, o_bav1))
    bufs = (b0, b1, b2)
    sems = (s0, s1, s2)
    cps = [None, None, None]
    for i, (tab, idx, out) in enumerate(work):
        k = i % 3
        if cps[k] is not None:
            cps[k].wait()
            pltpu.sync_copy(bufs[k], work[i - 3][2].at[pl.ds(base, _BPW)])
        cps[k] = pltpu.async_copy(tab.at[idx], bufs[k], sems[k])
    for i in (6, 7, 8):
        k = i % 3
        cps[k].wait()
        pltpu.sync_copy(bufs[k], work[i][2].at[pl.ds(base, _BPW)])


_row_out = jax.ShapeDtypeStruct((_BATCH, 128), jnp.float32)
_gather_call = pl.kernel(
    _gather_body,
    out_type=(_row_out,) * 9,
    mesh=_sc_mesh,
    scratch_types=[pltpu.VMEM((_BPW,), jnp.int32)] * 5
                  + [pltpu.VMEM((_BPW, 128), jnp.float32)] * 3
                  + [pltpu.SemaphoreType.DMA] * 3,
    compiler_params=pltpu.CompilerParams(use_tc_tiling_on_sc=True),
)


_BM = 512


def _select_win(cat, off, offsets, width):
    out = cat[:, offsets[0]:offsets[0] + width]
    for o in offsets[1:]:
        out = jnp.where(off == o, cat[:, o:o + width], out)
    return out


def _tc_body(x_ref, w1_ref, b1_ref, w2_ref, b2_ref, w3_ref, b3_ref, t_ref,
             jp_ref, jv_ref, bq_ref, bp0_ref, bp1_ref, blv0_ref, blv1_ref,
             bav0_ref, bav1_ref,
             act_ref, ojp_ref, ojv_ref, obp_ref, obq_ref, oblv_ref, obav_ref):
    h = jnp.dot(x_ref[...], w1_ref[...],
                preferred_element_type=jnp.float32) + b1_ref[...]
    h = jnp.where(h > 0, h, jnp.exp(h) - 1.0)
    h = jnp.dot(h, w2_ref[...], preferred_element_type=jnp.float32) + b2_ref[...]
    h = jnp.where(h > 0, h, jnp.exp(h) - 1.0)
    act_ref[...] = jnp.dot(h, w3_ref[...],
                           preferred_element_type=jnp.float32) + b3_ref[...]

    t = jnp.minimum(t_ref[...], _T_TOTAL - 1)
    off_a = (t & 3) << 5
    ojp_ref[...] = _select_win(jp_ref[...], off_a, (0, 32, 64, 96), _NJ)
    ojv_ref[...] = _select_win(jv_ref[...], off_a, (0, 32, 64, 96), _NJ)
    off_q = (t & 1) << 6
    obq_ref[...] = _select_win(bq_ref[...], off_q, (0, 64), _NB * 4)
    off_b = (t * 48) & 127
    offs_b = tuple(range(0, 128, 16))
    for r0, r1, o in ((bp0_ref, bp1_ref, obp_ref),
                      (blv0_ref, blv1_ref, oblv_ref),
                      (bav0_ref, bav1_ref, obav_ref)):
        cat = jnp.concatenate([r0[...], r1[...]], axis=1)
        o[...] = _select_win(cat, off_b, offs_b, _NB * 3)


_row_spec = pl.BlockSpec((_BM, 128), lambda i: (i, 0))
_tc_call = pl.pallas_call(
    _tc_body,
    grid=(_BATCH // _BM,),
    in_specs=[
        pl.BlockSpec((_BM, _OBS), lambda i: (i, 0)),
        pl.BlockSpec((_OBS, _H1), lambda i: (0, 0)),
        pl.BlockSpec((1, _H1), lambda i: (0, 0)),
        pl.BlockSpec((_H1, _H2), lambda i: (0, 0)),
        pl.BlockSpec((1, _H2), lambda i: (0, 0)),
        pl.BlockSpec((_H2, _ACT), lambda i: (0, 0)),
        pl.BlockSpec((1, _ACT), lambda i: (0, 0)),
        pl.BlockSpec((_BM, 1), lambda i: (i, 0)),
    ] + [_row_spec] * 9,
    out_specs=[
        pl.BlockSpec((_BM, _ACT), lambda i: (i, 0)),
        pl.BlockSpec((_BM, _NJ), lambda i: (i, 0)),
        pl.BlockSpec((_BM, _NJ), lambda i: (i, 0)),
        pl.BlockSpec((_BM, _NB * 3), lambda i: (i, 0)),
        pl.BlockSpec((_BM, _NB * 4), lambda i: (i, 0)),
        pl.BlockSpec((_BM, _NB * 3), lambda i: (i, 0)),
        pl.BlockSpec((_BM, _NB * 3), lambda i: (i, 0)),
    ],
    out_shape=[
        jax.ShapeDtypeStruct((_BATCH, _ACT), jnp.float32),
        jax.ShapeDtypeStruct((_BATCH, _NJ), jnp.float32),
        jax.ShapeDtypeStruct((_BATCH, _NJ), jnp.float32),
        jax.ShapeDtypeStruct((_BATCH, _NB * 3), jnp.float32),
        jax.ShapeDtypeStruct((_BATCH, _NB * 4), jnp.float32),
        jax.ShapeDtypeStruct((_BATCH, _NB * 3), jnp.float32),
        jax.ShapeDtypeStruct((_BATCH, _NB * 3), jnp.float32),
    ],
)


def kernel(x, time_step, W1, b1, W2, b2, W3, b3,
           joint_pos, joint_vel, body_pos_w, body_quat_w,
           body_lin_vel_w, body_ang_vel_w):
    ts = time_step.astype(jnp.int32).reshape(_BATCH)
    rows = _gather_call(
        ts,
        joint_pos.reshape(_VA, 128),
        joint_vel.reshape(_VA, 128),
        body_quat_w.reshape(_VQ, 128),
        body_pos_w.reshape(_VB, 128),
        body_lin_vel_w.reshape(_VB, 128),
        body_ang_vel_w.reshape(_VB, 128),
    )
    actions, g_jp, g_jv, g_bp, g_bq, g_blv, g_bav = _tc_call(
        x, W1, b1.reshape(1, _H1), W2, b2.reshape(1, _H2),
        W3, b3.reshape(1, _ACT), ts.reshape(_BATCH, 1), *rows)
    return (actions,
            g_jp,
            g_jv,
            g_bp.reshape(_BATCH, _NB, 3),
            g_bq.reshape(_BATCH, _NB, 4),
            g_blv.reshape(_BATCH, _NB, 3),
            g_bav.reshape(_BATCH, _NB, 3))


# speedup vs baseline: 14.2506x; 14.2506x over previous
import jax
import jax.numpy as jnp
from jax import lax
from jax.experimental import pallas as pl
from jax.experimental.pallas import tpu as pltpu
from jax.experimental.pallas import tpu_sc as plsc

_T_TOTAL = 100000
_OBS = 512
_H1 = 256
_H2 = 128
_ACT = 32
_NJ = 32
_NB = 16
_BATCH = 4096

_NC, _NS, _L = 2, 16, 16
_NW = _NC * _NS
_BPW = _BATCH // _NW

_VA = _T_TOTAL * _NJ // 128

_DIMS = (_NB * 4, _NB * 3, _NB * 3, _NB * 3)

_sc_mesh = plsc.VectorSubcoreMesh(
    core_axis_name="c", subcore_axis_name="s",
    num_cores=_NC, num_subcores=_NS)


def _joint_body(ts_hbm, jp, jv, o_jp, o_jv, t_v, ra_v, b0, b1, s0, s1):
    wid = lax.axis_index("s") * _NC + lax.axis_index("c")
    base = wid * _BPW
    pltpu.sync_copy(ts_hbm.at[pl.ds(base, _BPW)], t_v)
    for i in range(_BPW // _L):
        sl = pl.ds(i * _L, _L)
        ra_v[sl] = jnp.minimum(t_v[sl], _T_TOTAL - 1) >> 2
    cp0 = pltpu.async_copy(jp.at[ra_v], b0, s0)
    cp1 = pltpu.async_copy(jv.at[ra_v], b1, s1)
    cp0.wait()
    pltpu.sync_copy(b0, o_jp.at[pl.ds(base, _BPW)])
    cp1.wait()
    pltpu.sync_copy(b1, o_jv.at[pl.ds(base, _BPW)])


_joint_call = pl.kernel(
    _joint_body,
    out_type=(jax.ShapeDtypeStruct((_BATCH, 128), jnp.float32),) * 2,
    mesh=_sc_mesh,
    scratch_types=[pltpu.VMEM((_BPW,), jnp.int32)] * 2
                  + [pltpu.VMEM((_BPW, 128), jnp.float32)] * 2
                  + [pltpu.SemaphoreType.DMA] * 2,
    compiler_params=pltpu.CompilerParams(use_tc_tiling_on_sc=True),
)


def _body_gather(ts_hbm, t0, t1, t2, t3, o0, o1, o2, o3,
                 idx_v, b0, b1, b2, b3, sem):
    wid = lax.axis_index("s") * _NC + lax.axis_index("c")
    base = wid * _BPW
    pltpu.sync_copy(ts_hbm.at[pl.ds(base, _BPW)], idx_v)
    for i in range(_BPW // _L):
        sl = pl.ds(i * _L, _L)
        idx_v[sl] = jnp.minimum(idx_v[sl], _T_TOTAL - 1)
    tabs = (t0, t1, t2, t3)
    bufs = (b0, b1, b2, b3)
    outs = (o0, o1, o2, o3)
    cps = [pltpu.async_copy(tab.at[idx_v], buf, sem)
           for tab, buf in zip(tabs, bufs)]
    for cp, buf, out in zip(cps, bufs, outs):
        cp.wait()
        pltpu.sync_copy(buf, out.at[pl.ds(base, _BPW)])


_body_call = pl.kernel(
    _body_gather,
    out_type=tuple(jax.ShapeDtypeStruct((_BATCH, d), jnp.float32)
                   for d in _DIMS),
    mesh=_sc_mesh,
    scratch_types=[pltpu.VMEM((_BPW,), jnp.int32)]
                  + [pltpu.VMEM((_BPW, d), jnp.float32) for d in _DIMS]
                  + [pltpu.SemaphoreType.DMA],
    compiler_params=pltpu.CompilerParams(use_tc_tiling_on_sc=False),
)


_BM = 512


def _select_win(cat, off, offsets, width):
    out = cat[:, offsets[0]:offsets[0] + width]
    for o in offsets[1:]:
        out = jnp.where(off == o, cat[:, o:o + width], out)
    return out


def _tc_body(x_ref, w1_ref, b1_ref, w2_ref, b2_ref, w3_ref, b3_ref, t_ref,
             jp_ref, jv_ref, act_ref, ojp_ref, ojv_ref):
    h = jnp.dot(x_ref[...], w1_ref[...],
                preferred_element_type=jnp.float32) + b1_ref[...]
    h = jnp.where(h > 0, h, jnp.exp(h) - 1.0)
    h = jnp.dot(h, w2_ref[...], preferred_element_type=jnp.float32) + b2_ref[...]
    h = jnp.where(h > 0, h, jnp.exp(h) - 1.0)
    act_ref[...] = jnp.dot(h, w3_ref[...],
                           preferred_element_type=jnp.float32) + b3_ref[...]

    t = jnp.minimum(t_ref[...], _T_TOTAL - 1)
    off_a = (t & 3) << 5
    ojp_ref[...] = _select_win(jp_ref[...], off_a, (0, 32, 64, 96), _NJ)
    ojv_ref[...] = _select_win(jv_ref[...], off_a, (0, 32, 64, 96), _NJ)


_row_spec = pl.BlockSpec((_BM, 128), lambda i: (i, 0))
_tc_call = pl.pallas_call(
    _tc_body,
    grid=(_BATCH // _BM,),
    in_specs=[
        pl.BlockSpec((_BM, _OBS), lambda i: (i, 0)),
        pl.BlockSpec((_OBS, _H1), lambda i: (0, 0)),
        pl.BlockSpec((1, _H1), lambda i: (0, 0)),
        pl.BlockSpec((_H1, _H2), lambda i: (0, 0)),
        pl.BlockSpec((1, _H2), lambda i: (0, 0)),
        pl.BlockSpec((_H2, _ACT), lambda i: (0, 0)),
        pl.BlockSpec((1, _ACT), lambda i: (0, 0)),
        pl.BlockSpec((_BM, 1), lambda i: (i, 0)),
        _row_spec, _row_spec,
    ],
    out_specs=[
        pl.BlockSpec((_BM, _ACT), lambda i: (i, 0)),
        pl.BlockSpec((_BM, _NJ), lambda i: (i, 0)),
        pl.BlockSpec((_BM, _NJ), lambda i: (i, 0)),
    ],
    out_shape=[
        jax.ShapeDtypeStruct((_BATCH, _ACT), jnp.float32),
        jax.ShapeDtypeStruct((_BATCH, _NJ), jnp.float32),
        jax.ShapeDtypeStruct((_BATCH, _NJ), jnp.float32),
    ],
)


def kernel(x, time_step, W1, b1, W2, b2, W3, b3,
           joint_pos, joint_vel, body_pos_w, body_quat_w,
           body_lin_vel_w, body_ang_vel_w):
    ts = time_step.astype(jnp.int32).reshape(_BATCH)
    jp_rows, jv_rows = _joint_call(
        ts,
        joint_pos.reshape(_VA, 128),
        joint_vel.reshape(_VA, 128),
    )
    g_bq, g_bp, g_blv, g_bav = _body_call(
        ts,
        body_quat_w.reshape(_T_TOTAL, _NB * 4),
        body_pos_w.reshape(_T_TOTAL, _NB * 3),
        body_lin_vel_w.reshape(_T_TOTAL, _NB * 3),
        body_ang_vel_w.reshape(_T_TOTAL, _NB * 3),
    )
    actions, g_jp, g_jv = _tc_call(
        x, W1, b1.reshape(1, _H1), W2, b2.reshape(1, _H2),
        W3, b3.reshape(1, _ACT), ts.reshape(_BATCH, 1), jp_rows, jv_rows)
    return (actions,
            g_jp,
            g_jv,
            g_bp.reshape(_BATCH, _NB, 3),
            g_bq.reshape(_BATCH, _NB, 4),
            g_blv.reshape(_BATCH, _NB, 3),
            g_bav.reshape(_BATCH, _NB, 3))
